# baseline (speedup 1.0000x reference)
import jax
import jax.numpy as jnp
from jax import lax
from jax.experimental import pallas as pl
from jax.experimental.pallas import tpu as pltpu
from jax.experimental.pallas import tpu_sc as plsc

_L = 16
_NC = 2
_NS = 16


def _make_sc_avg(S, D, lo):
    C = D // _L
    nw = _NC * _NS
    scale = 1.0 / S

    def compute(buf, acc):
        def _init(r, cs):
            c0, c1, c2, c3 = cs
            b = (S - 1) + 4 * r
            return (
                c0 + buf[b, :],
                c1 + buf[b + 1, :],
                c2 + buf[b + 2, :],
                c3 + buf[b + 3, :],
            )

        z = jnp.zeros((_L,), jnp.float32)
        c0, c1, c2, c3 = lax.fori_loop(0, S // 4, _init, (z, z, z, z), unroll=8)
        w = (c0 + c1) + (c2 + c3)
        acc[0, :] = w * scale

        def _slide4(j, w):
            i = 4 * j
            d1 = buf[(S - 2) - i, :] - buf[(2 * S - 2) - i, :]
            d2 = buf[(S - 3) - i, :] - buf[(2 * S - 3) - i, :]
            d3 = buf[(S - 4) - i, :] - buf[(2 * S - 4) - i, :]
            d4 = buf[(S - 5) - i, :] - buf[(2 * S - 5) - i, :]
            s2 = d1 + d2
            s34 = d3 + d4
            w1 = w + d1
            w2 = w + s2
            w3 = w2 + d3
            w4 = w + (s2 + s34)
            acc[i + 1, :] = w1 * scale
            acc[i + 2, :] = w2 * scale
            acc[i + 3, :] = w3 * scale
            acc[i + 4, :] = w4 * scale
            return w4

        w = lax.fori_loop(0, S // 4 - 1, _slide4, w, unroll=4)

        def _slide1(i, w):
            w = w + buf[(S - 1) - i, :] - buf[(2 * S - 1) - i, :]
            acc[i, :] = w * scale
            return w

        lax.fori_loop(S - 3, S, _slide1, w, unroll=3)

    def body(table_hbm, avg_hbm, buf0, buf1, acc0, acc1, si0, si1, so0, so1):
        wid = lax.axis_index("s") * _NC + lax.axis_index("c")
        chunk0 = wid
        chunk1 = wid + nw
        has1 = chunk1 < C

        def in_cp(chunk, buf, sem):
            return pltpu.make_async_copy(
                table_hbm.at[pl.ds(lo, 2 * S - 1), pl.ds(chunk * _L, _L)],
                buf,
                sem,
            )

        def out_cp(chunk, acc, sem):
            return pltpu.make_async_copy(
                acc, avg_hbm.at[:, pl.ds(chunk * _L, _L)], sem
            )

        in_cp(chunk0, buf0, si0).start()

        @pl.when(has1)
        def _():
            in_cp(chunk1, buf1, si1).start()

        in_cp(chunk0, buf0, si0).wait()
        compute(buf0, acc0)
        out_cp(chunk0, acc0, so0).start()

        @pl.when(has1)
        def _():
            in_cp(chunk1, buf1, si1).wait()
            compute(buf1, acc1)
            out_cp(chunk1, acc1, so1).start()

        out_cp(chunk0, acc0, so0).wait()

        @pl.when(has1)
        def _():
            out_cp(chunk1, acc1, so1).wait()

    return pl.kernel(
        body,
        out_type=jax.ShapeDtypeStruct((S, D), jnp.float32),
        mesh=plsc.VectorSubcoreMesh(core_axis_name="c", subcore_axis_name="s"),
        scratch_types=[
            pltpu.VMEM((2 * S - 1, _L), jnp.float32),
            pltpu.VMEM((2 * S - 1, _L), jnp.float32),
            pltpu.VMEM((S, _L), jnp.float32),
            pltpu.VMEM((S, _L), jnp.float32),
            pltpu.SemaphoreType.DMA,
            pltpu.SemaphoreType.DMA,
            pltpu.SemaphoreType.DMA,
            pltpu.SemaphoreType.DMA,
        ],
        compiler_params=pltpu.CompilerParams(use_tc_tiling_on_sc=False),
    )


def _add_body(avg_ref, x_ref, o_ref):
    o_ref[...] = x_ref[...] + avg_ref[...][None]


def kernel(x, rel_table):
    B, S, D = x.shape
    max_len = (rel_table.shape[0] + 1) // 2
    lo = max_len - S

    avg = _make_sc_avg(S, D, lo)(rel_table)

    SB = S // 4
    return pl.pallas_call(
        _add_body,
        grid=(B, 4),
        in_specs=[
            pl.BlockSpec((SB, D), lambda b, s: (s, 0)),
            pl.BlockSpec((1, SB, D), lambda b, s: (b, s, 0)),
        ],
        out_specs=pl.BlockSpec((1, SB, D), lambda b, s: (b, s, 0)),
        out_shape=jax.ShapeDtypeStruct((B, S, D), jnp.float32),
    )(avg, x)


# speedup vs baseline: 1.1132x; 1.1132x over previous
import jax
import jax.numpy as jnp
from jax import lax
from jax.experimental import pallas as pl
from jax.experimental.pallas import tpu as pltpu
from jax.experimental.pallas import tpu_sc as plsc

_L = 16
_NC = 2
_NS = 16


def _make_sc_avg(S, D, lo):
    C = D // _L
    nw = _NC * _NS
    scale = 1.0 / S

    def compute(buf, acc):
        def _init(r, cs):
            c0, c1, c2, c3 = cs
            b = (S - 1) + 4 * r
            return (
                c0 + buf[b, :],
                c1 + buf[b + 1, :],
                c2 + buf[b + 2, :],
                c3 + buf[b + 3, :],
            )

        z = jnp.zeros((_L,), jnp.float32)
        c0, c1, c2, c3 = lax.fori_loop(0, S // 4, _init, (z, z, z, z), unroll=8)
        w = (c0 + c1) + (c2 + c3)
        acc[0, :] = w * scale

        def _slide4(j, w):
            i = 4 * j
            d1 = buf[(S - 2) - i, :] - buf[(2 * S - 2) - i, :]
            d2 = buf[(S - 3) - i, :] - buf[(2 * S - 3) - i, :]
            d3 = buf[(S - 4) - i, :] - buf[(2 * S - 4) - i, :]
            d4 = buf[(S - 5) - i, :] - buf[(2 * S - 5) - i, :]
            s2 = d1 + d2
            s34 = d3 + d4
            w1 = w + d1
            w2 = w + s2
            w3 = w2 + d3
            w4 = w + (s2 + s34)
            acc[i + 1, :] = w1 * scale
            acc[i + 2, :] = w2 * scale
            acc[i + 3, :] = w3 * scale
            acc[i + 4, :] = w4 * scale
            return w4

        w = lax.fori_loop(0, S // 4 - 1, _slide4, w, unroll=4)

        def _slide1(i, w):
            w = w + buf[(S - 1) - i, :] - buf[(2 * S - 1) - i, :]
            acc[i, :] = w * scale
            return w

        lax.fori_loop(S - 3, S, _slide1, w, unroll=3)

    def body(table_hbm, avg_hbm, buf0, buf1, acc0, acc1, si0, si1, so0, so1):
        wid = lax.axis_index("s") * _NC + lax.axis_index("c")
        chunk0 = wid
        chunk1 = wid + nw
        has1 = chunk1 < C

        def in_cp(chunk, buf, sem):
            return pltpu.make_async_copy(
                table_hbm.at[pl.ds(lo, 2 * S - 1), pl.ds(chunk * _L, _L)],
                buf,
                sem,
            )

        def out_cp(chunk, acc, sem):
            return pltpu.make_async_copy(
                acc, avg_hbm.at[:, pl.ds(chunk * _L, _L)], sem
            )

        in_cp(chunk0, buf0, si0).start()

        @pl.when(has1)
        def _():
            in_cp(chunk1, buf1, si1).start()

        in_cp(chunk0, buf0, si0).wait()
        compute(buf0, acc0)
        out_cp(chunk0, acc0, so0).start()

        @pl.when(has1)
        def _():
            in_cp(chunk1, buf1, si1).wait()
            compute(buf1, acc1)
            out_cp(chunk1, acc1, so1).start()

        out_cp(chunk0, acc0, so0).wait()

        @pl.when(has1)
        def _():
            out_cp(chunk1, acc1, so1).wait()

    return pl.kernel(
        body,
        out_type=jax.ShapeDtypeStruct((S, D), jnp.float32),
        mesh=plsc.VectorSubcoreMesh(core_axis_name="c", subcore_axis_name="s"),
        scratch_types=[
            pltpu.VMEM((2 * S - 1, _L), jnp.float32),
            pltpu.VMEM((2 * S - 1, _L), jnp.float32),
            pltpu.VMEM((S, _L), jnp.float32),
            pltpu.VMEM((S, _L), jnp.float32),
            pltpu.SemaphoreType.DMA,
            pltpu.SemaphoreType.DMA,
            pltpu.SemaphoreType.DMA,
            pltpu.SemaphoreType.DMA,
        ],
        compiler_params=pltpu.CompilerParams(use_tc_tiling_on_sc=False),
    )


def _add_body(avg_ref, x_ref, o_ref):
    o_ref[...] = x_ref[...] + avg_ref[...][None]


def kernel(x, rel_table):
    B, S, D = x.shape
    max_len = (rel_table.shape[0] + 1) // 2
    lo = max_len - S

    avg = _make_sc_avg(S, D, lo)(rel_table)

    return pl.pallas_call(
        _add_body,
        grid=(B,),
        in_specs=[
            pl.BlockSpec((S, D), lambda b: (0, 0)),
            pl.BlockSpec((1, S, D), lambda b: (b, 0, 0)),
        ],
        out_specs=pl.BlockSpec((1, S, D), lambda b: (b, 0, 0)),
        out_shape=jax.ShapeDtypeStruct((B, S, D), jnp.float32),
    )(avg, x)
